# baseline (speedup 1.0000x reference)
import functools

import jax
import jax.numpy as jnp
from jax import lax
from jax.experimental import pallas as pl
from jax.experimental.pallas import tpu as pltpu
from jax.experimental.pallas import tpu_sc as plsc

_INFO = plsc.get_sparse_core_info()
_NC = _INFO.num_cores
_NS = _INFO.num_subcores
_NW = _NC * _NS
_LANES = 16

_CHUNK = 128
_CHUNKS_PER_W = 8
_PER_W = _CHUNK * _CHUNKS_PER_W
_MP = _NW * _PER_W


def _sc_loss_kernel(pred_flat, idx_s, idx_e, tgt, msk):
    mesh = plsc.VectorSubcoreMesh(core_axis_name="c", subcore_axis_name="s")

    @functools.partial(
        pl.kernel,
        out_type=jax.ShapeDtypeStruct((_NW, 2 * _LANES), jnp.float32),
        mesh=mesh,
        scratch_types=[
            pltpu.VMEM((_CHUNKS_PER_W, _CHUNK), jnp.int32),
            pltpu.VMEM((_CHUNKS_PER_W, _CHUNK), jnp.int32),
            pltpu.VMEM((_PER_W,), jnp.float32),
            pltpu.VMEM((_PER_W,), jnp.float32),
            pltpu.VMEM((_PER_W,), jnp.float32),
            pltpu.VMEM((_PER_W,), jnp.float32),
            pltpu.VMEM((2 * _LANES,), jnp.float32),
            pltpu.SemaphoreType.DMA,
        ],
    )
    def body(pred_hbm, idx_s_hbm, idx_e_hbm, tgt_hbm, msk_hbm, out_hbm,
             idx_s_v, idx_e_v, tgt_v, msk_v, vs_v, ve_v, out_v, sem):
        wid = lax.axis_index("s") * _NC + lax.axis_index("c")

        pltpu.sync_copy(idx_s_hbm.at[wid], idx_s_v)
        pltpu.sync_copy(idx_e_hbm.at[wid], idx_e_v)
        pltpu.sync_copy(tgt_hbm.at[wid], tgt_v)
        pltpu.sync_copy(msk_hbm.at[wid], msk_v)

        copies = []
        for j in range(_CHUNKS_PER_W):
            dst = pl.ds(j * _CHUNK, _CHUNK)
            copies.append(
                pltpu.async_copy(pred_hbm.at[idx_s_v.at[j]], vs_v.at[dst], sem))
            copies.append(
                pltpu.async_copy(pred_hbm.at[idx_e_v.at[j]], ve_v.at[dst], sem))
        for cp in copies:
            cp.wait()

        acc = jnp.zeros((_LANES,), jnp.float32)
        mac = jnp.zeros((_LANES,), jnp.float32)
        for q in range(_PER_W // _LANES):
            sl = pl.ds(q * _LANES, _LANES)
            g = (vs_v[sl] + ve_v[sl]) * 0.5
            m = msk_v[sl]
            d = g * m - tgt_v[sl] * m
            ad = jnp.abs(d)
            l = jnp.where(ad < 1.0, 0.5 * d * d, ad - 0.5)
            acc = acc + l
            mac = mac + m
        out_v[pl.ds(0, _LANES)] = acc
        out_v[pl.ds(_LANES, _LANES)] = mac
        pltpu.sync_copy(out_v, out_hbm.at[wid])

    return body(pred_flat, idx_s, idx_e, tgt, msk)


def kernel(pred, mask, ind, target):
    B, C, H, W = pred.shape
    K = ind.shape[1]
    HW = H * W
    M = B * K * C

    pred_flat = pred.reshape(-1)

    base = (jnp.arange(B, dtype=jnp.int32)[:, None, None] * C
            + jnp.arange(C, dtype=jnp.int32)[None, None, :]) * HW
    idx_s = (base + ind[:, :, 0][:, :, None]).reshape(-1)
    idx_e = (base + ind[:, :, 1][:, :, None]).reshape(-1)
    tgt = target.reshape(-1)
    msk = jnp.broadcast_to(mask[:, :, None], (B, K, C)).reshape(-1)

    pad = _MP - M
    idx_s = jnp.pad(idx_s, (0, pad)).reshape(_NW, _CHUNKS_PER_W, _CHUNK)
    idx_e = jnp.pad(idx_e, (0, pad)).reshape(_NW, _CHUNKS_PER_W, _CHUNK)
    tgt = jnp.pad(tgt, (0, pad)).reshape(_NW, _PER_W)
    msk = jnp.pad(msk, (0, pad)).reshape(_NW, _PER_W)

    out = _sc_loss_kernel(pred_flat, idx_s, idx_e, tgt, msk)
    out = out.reshape(_NW, 2, _LANES)
    num = jnp.sum(out[:, 0, :])
    den = jnp.sum(out[:, 1, :])
    return num / (den + 0.0001)


# speedup vs baseline: 2.9884x; 2.9884x over previous
import functools

import jax
import jax.numpy as jnp
from jax import lax
from jax.experimental import pallas as pl
from jax.experimental.pallas import tpu as pltpu
from jax.experimental.pallas import tpu_sc as plsc

_INFO = plsc.get_sparse_core_info()
_NC = _INFO.num_cores
_NS = _INFO.num_subcores
_NW = _NC * _NS
_L = 16


def _sc_loss_kernel(pred_lin, ind_pk, fm_pk, B, C, H, W, K, NT):
    HW = H * W
    KG = NT // _L
    mesh = plsc.VectorSubcoreMesh(core_axis_name="c", subcore_axis_name="s")

    @functools.partial(
        pl.kernel,
        out_type=jax.ShapeDtypeStruct((_NW, 2 * _L), jnp.float32),
        mesh=mesh,
        scratch_types=[
            pltpu.VMEM((2 * NT,), jnp.int32),
            pltpu.VMEM((3 * NT,), jnp.float32),
            pltpu.VMEM((4 * NT,), jnp.int32),
            pltpu.VMEM((4 * NT,), jnp.float32),
            pltpu.VMEM((2 * _L,), jnp.float32),
            pltpu.SemaphoreType.DMA,
            pltpu.SemaphoreType.DMA,
        ],
    )
    def body(pred_hbm, ind_hbm, fm_hbm, out_hbm,
             ind_v, fm_v, idx_v, gv_v, out_v, gsem, isem):
        wid = lax.axis_index("s") * _NC + lax.axis_index("c")

        cp_fm = pltpu.async_copy(fm_hbm.at[wid], fm_v, isem)
        pltpu.sync_copy(ind_hbm.at[wid], ind_v)

        base = wid * (C * HW)

        copies = []
        for g in range(0, KG, 2):
            cbase = (g // 2) * 128
            for kind in (0, 1):
                for u in (0, 1):
                    p = ind_v[pl.ds(kind * NT + (g + u) * _L, _L)]
                    h = p >> 8
                    w = p & (W - 1)
                    tiled = (((h >> 3) << 11) + ((w >> 7) << 10)
                             + ((h & 7) << 7) + (w & 127))
                    a0 = base + tiled
                    koff = cbase + kind * 64 + u * _L
                    idx_v[pl.ds(koff, _L)] = a0
                    idx_v[pl.ds(koff + 2 * _L, _L)] = a0 + HW
            sl = pl.ds(cbase, 128)
            copies.append(pltpu.async_copy(pred_hbm.at[idx_v.at[sl]],
                                           gv_v.at[sl], gsem))
        cp_fm.wait()
        for cp in copies:
            cp.wait()

        acc = jnp.zeros((_L,), jnp.float32)
        mac = jnp.zeros((_L,), jnp.float32)
        for g in range(KG):
            cbase = (g // 2) * 128
            u = g % 2
            m = fm_v[pl.ds(2 * NT + g * _L, _L)]
            for c in (0, 1):
                vs = gv_v[pl.ds(cbase + c * 2 * _L + u * _L, _L)]
                ve = gv_v[pl.ds(cbase + 64 + c * 2 * _L + u * _L, _L)]
                t = fm_v[pl.ds(c * NT + g * _L, _L)]
                gavg = (vs + ve) * 0.5
                d = gavg * m - t * m
                ad = jnp.abs(d)
                l = jnp.where(ad < 1.0, 0.5 * d * d, ad - 0.5)
                acc = acc + l
                mac = mac + m
        out_v[pl.ds(0, _L)] = acc
        out_v[pl.ds(_L, _L)] = mac
        pltpu.sync_copy(out_v, out_hbm.at[wid])

    return body(pred_lin, ind_pk, fm_pk)


def kernel(pred, mask, ind, target):
    B, C, H, W = pred.shape
    K = ind.shape[1]
    NT = ((K + _L - 1) // _L + 7) // 8 * 8 * _L
    pk = NT - K

    pred_lin = (pred.reshape(B, C, H // 8, 8, W // 128, 128)
                .swapaxes(3, 4).reshape(-1))

    ind_pk = jnp.pad(ind.swapaxes(1, 2),
                     ((0, 0), (0, 0), (0, pk))).reshape(B, 2 * NT)
    fm_pk = jnp.concatenate(
        [jnp.pad(target.swapaxes(1, 2), ((0, 0), (0, 0), (0, pk))),
         jnp.pad(mask, ((0, 0), (0, pk)))[:, None, :]],
        axis=1).reshape(B, 3 * NT)

    out = _sc_loss_kernel(pred_lin, ind_pk, fm_pk, B, C, H, W, K, NT)
    out = out.reshape(_NW, 2, _L)
    num = jnp.sum(out[:, 0, :])
    den = jnp.sum(out[:, 1, :])
    return num / (den + 0.0001)


# speedup vs baseline: 3.0114x; 1.0077x over previous
import functools

import jax
import jax.numpy as jnp
from jax import lax
from jax.experimental import pallas as pl
from jax.experimental.pallas import tpu as pltpu
from jax.experimental.pallas import tpu_sc as plsc

_INFO = plsc.get_sparse_core_info()
_NC = _INFO.num_cores
_NS = _INFO.num_subcores
_NW = _NC * _NS
_L = 16


def _sc_loss_kernel(pred_lin, pk, B, C, H, W, K, NT):
    HW = H * W
    KG = NT // _L
    mesh = plsc.VectorSubcoreMesh(core_axis_name="c", subcore_axis_name="s")

    @functools.partial(
        pl.kernel,
        out_type=jax.ShapeDtypeStruct((_NW, 2 * _L), jnp.float32),
        mesh=mesh,
        scratch_types=[
            pltpu.VMEM((5 * NT,), jnp.float32),
            pltpu.VMEM((4 * NT,), jnp.int32),
            pltpu.VMEM((4 * NT,), jnp.float32),
            pltpu.VMEM((2 * _L,), jnp.float32),
            pltpu.SemaphoreType.DMA,
        ],
    )
    def body(pred_hbm, pk_hbm, out_hbm, pk_v, idx_v, gv_v, out_v, gsem):
        wid = lax.axis_index("s") * _NC + lax.axis_index("c")

        pltpu.sync_copy(pk_hbm.at[wid], pk_v)

        base = wid * (C * HW)

        copies = []
        for g in range(0, KG, 2):
            cbase = (g // 2) * 128
            for kind in (0, 1):
                for u in (0, 1):
                    p = pk_v[pl.ds(kind * NT + (g + u) * _L, _L)
                             ].astype(jnp.int32)
                    h = p >> 8
                    w = p & (W - 1)
                    tiled = (((h >> 3) << 11) + ((w >> 7) << 10)
                             + ((h & 7) << 7) + (w & 127))
                    a0 = base + tiled
                    koff = cbase + kind * 64 + u * _L
                    idx_v[pl.ds(koff, _L)] = a0
                    idx_v[pl.ds(koff + 2 * _L, _L)] = a0 + HW
            sl = pl.ds(cbase, 128)
            copies.append(pltpu.async_copy(pred_hbm.at[idx_v.at[sl]],
                                           gv_v.at[sl], gsem))
        for cp in copies:
            cp.wait()

        acc = jnp.zeros((_L,), jnp.float32)
        mac = jnp.zeros((_L,), jnp.float32)
        for g in range(KG):
            cbase = (g // 2) * 128
            u = g % 2
            m = pk_v[pl.ds(4 * NT + g * _L, _L)]
            for c in (0, 1):
                vs = gv_v[pl.ds(cbase + c * 2 * _L + u * _L, _L)]
                ve = gv_v[pl.ds(cbase + 64 + c * 2 * _L + u * _L, _L)]
                t = pk_v[pl.ds((2 + c) * NT + g * _L, _L)]
                gavg = (vs + ve) * 0.5
                d = gavg * m - t * m
                ad = jnp.abs(d)
                l = jnp.where(ad < 1.0, 0.5 * d * d, ad - 0.5)
                acc = acc + l
                mac = mac + m
        out_v[pl.ds(0, _L)] = acc
        out_v[pl.ds(_L, _L)] = mac
        pltpu.sync_copy(out_v, out_hbm.at[wid])

    return body(pred_lin, pk)


def kernel(pred, mask, ind, target):
    B, C, H, W = pred.shape
    K = ind.shape[1]
    NT = ((K + _L - 1) // _L + 7) // 8 * 8 * _L
    pk = NT - K

    pred_lin = (pred.reshape(B, C, H // 8, 8, W // 128, 128)
                .swapaxes(3, 4).reshape(-1))

    pk_op = jnp.concatenate(
        [jnp.pad(ind.astype(jnp.float32).swapaxes(1, 2),
                 ((0, 0), (0, 0), (0, pk))),
         jnp.pad(target.swapaxes(1, 2), ((0, 0), (0, 0), (0, pk))),
         jnp.pad(mask, ((0, 0), (0, pk)))[:, None, :]],
        axis=1).reshape(B, 5 * NT)

    out = _sc_loss_kernel(pred_lin, pk_op, B, C, H, W, K, NT)
    out = out.reshape(_NW, 2, _L)
    num = jnp.sum(out[:, 0, :])
    den = jnp.sum(out[:, 1, :])
    return num / (den + 0.0001)


# speedup vs baseline: 3.0715x; 1.0200x over previous
import functools

import jax
import jax.numpy as jnp
from jax import lax
from jax.experimental import pallas as pl
from jax.experimental.pallas import tpu as pltpu
from jax.experimental.pallas import tpu_sc as plsc

_INFO = plsc.get_sparse_core_info()
_NC = _INFO.num_cores
_NS = _INFO.num_subcores
_NW = _NC * _NS
_L = 16


def _sc_loss_kernel(pred_lin, pk, B, C, H, W, K, NT):
    HW = H * W
    KG = NT // _L
    mesh = plsc.VectorSubcoreMesh(core_axis_name="c", subcore_axis_name="s")

    @functools.partial(
        pl.kernel,
        out_type=jax.ShapeDtypeStruct((_NW, 2 * _L), jnp.float32),
        mesh=mesh,
        scratch_types=[
            pltpu.VMEM((5 * NT,), jnp.float32),
            pltpu.VMEM((4 * NT,), jnp.int32),
            pltpu.VMEM((4 * NT,), jnp.float32),
            pltpu.VMEM((2 * _L,), jnp.float32),
            pltpu.SemaphoreType.DMA,
        ],
    )
    def body(pred_hbm, pk_hbm, out_hbm, pk_v, idx_v, gv_v, out_v, gsem):
        wid = lax.axis_index("s") * _NC + lax.axis_index("c")

        pltpu.sync_copy(pk_hbm.at[wid], pk_v)

        base = wid * (C * HW)

        def build_chunk(j, _):
            cbase = j * 128
            for kind in (0, 1):
                for u in (0, 1):
                    p = pk_v[pl.ds(kind * NT + cbase // 4 + u * _L, _L)
                             ].astype(jnp.int32)
                    h = p >> 8
                    w = p & (W - 1)
                    tiled = (((h >> 3) << 11) + ((w >> 7) << 10)
                             + ((h & 7) << 7) + (w & 127))
                    a0 = base + tiled
                    koff = cbase + kind * 64 + u * _L
                    idx_v[pl.ds(koff, _L)] = a0
                    idx_v[pl.ds(koff + 2 * _L, _L)] = a0 + HW
            sl = pl.ds(cbase, 128)
            pltpu.async_copy(pred_hbm.at[idx_v.at[sl]], gv_v.at[sl], gsem)
            return _

        lax.fori_loop(0, KG // 2, build_chunk, None)
        pltpu.make_async_copy(pred_hbm.at[pl.ds(0, 4 * NT)], gv_v,
                              gsem).wait()

        def accum(g, carry):
            acc, mac = carry
            cbase = (g >> 1) * 128
            u = (g & 1) * _L
            m = pk_v[pl.ds(4 * NT + g * _L, _L)]
            for c in (0, 1):
                vs = gv_v[pl.ds(cbase + c * 2 * _L + u, _L)]
                ve = gv_v[pl.ds(cbase + 64 + c * 2 * _L + u, _L)]
                t = pk_v[pl.ds((2 + c) * NT + g * _L, _L)]
                gavg = (vs + ve) * 0.5
                d = gavg * m - t * m
                ad = jnp.abs(d)
                l = jnp.where(ad < 1.0, 0.5 * d * d, ad - 0.5)
                acc = acc + l
                mac = mac + m
            return acc, mac

        acc, mac = lax.fori_loop(
            0, KG, accum,
            (jnp.zeros((_L,), jnp.float32), jnp.zeros((_L,), jnp.float32)))
        out_v[pl.ds(0, _L)] = acc
        out_v[pl.ds(_L, _L)] = mac
        pltpu.sync_copy(out_v, out_hbm.at[wid])

    return body(pred_lin, pk)


def kernel(pred, mask, ind, target):
    B, C, H, W = pred.shape
    K = ind.shape[1]
    NT = ((K + _L - 1) // _L + 7) // 8 * 8 * _L
    pk = NT - K

    pred_lin = (pred.reshape(B, C, H // 8, 8, W // 128, 128)
                .swapaxes(3, 4).reshape(-1))

    pk_op = jnp.concatenate(
        [jnp.pad(ind.astype(jnp.float32).swapaxes(1, 2),
                 ((0, 0), (0, 0), (0, pk))),
         jnp.pad(target.swapaxes(1, 2), ((0, 0), (0, 0), (0, pk))),
         jnp.pad(mask, ((0, 0), (0, pk)))[:, None, :]],
        axis=1).reshape(B, 5 * NT)

    out = _sc_loss_kernel(pred_lin, pk_op, B, C, H, W, K, NT)
    out = out.reshape(_NW, 2, _L)
    num = jnp.sum(out[:, 0, :])
    den = jnp.sum(out[:, 1, :])
    return num / (den + 0.0001)
